# Initial kernel scaffold; baseline (speedup 1.0000x reference)
import jax
import jax.numpy as jnp
from jax.experimental import pallas as pl


def kernel(input0, input1, input2, input3, table0, table1, table2, table3):
    raise NotImplementedError("write your pallas kernel here")


# speedup vs baseline: 42.1728x; 42.1728x over previous
import jax
import jax.numpy as jnp
from jax.experimental import pallas as pl

_V = 1000
_E = 16
_B = 1024
_NC = _V * _E
_BB = 16


def _fill_kernel(i0, i1, i2, i3, r0, r1, r2, r3, o0, o1, o2, o3):
    col = jax.lax.broadcasted_iota(jnp.int32, (_BB, _NC), 1)
    for i_ref, r_ref, o_ref in ((i0, r0, o0), (i1, r1, o1), (i2, r2, o2), (i3, r3, o3)):
        hot = i_ref[...] * _E
        mask = (col >= hot) & (col < hot + _E)
        o_ref[...] = jnp.where(mask, r_ref[1:2, :], r_ref[0:1, :])


def kernel(input0, input1, input2, input3, table0, table1, table2, table3):
    inputs = (input0, input1, input2, input3)
    tables = (table0, table1, table2, table3)

    idx = [inp.reshape(_B, 1) for inp in inputs]
    rows = [jnp.stack([jnp.tile(t[0], _V), jnp.tile(t[1], _V)]) for t in tables]

    grid = (_B // _BB,)
    in_specs = (
        [pl.BlockSpec((_BB, 1), lambda i: (i, 0)) for _ in range(4)]
        + [pl.BlockSpec((2, _NC), lambda i: (0, 0)) for _ in range(4)]
    )
    out_specs = [pl.BlockSpec((_BB, _NC), lambda i: (i, 0)) for _ in range(4)]
    outs = pl.pallas_call(
        _fill_kernel,
        grid=grid,
        in_specs=in_specs,
        out_specs=out_specs,
        out_shape=[jax.ShapeDtypeStruct((_B, _NC), jnp.float32)] * 4,
    )(*idx, *rows)
    return tuple(o.reshape(_B, _V, _E) for o in outs)


# speedup vs baseline: 42.5939x; 1.0100x over previous
import jax
import jax.numpy as jnp
from jax.experimental import pallas as pl

_V = 1000
_E = 16
_B = 1024
_NC = _V * _E
_BB = 32


def _fill_kernel(i0, i1, i2, i3, r0, r1, r2, r3, o0, o1, o2, o3):
    colv = jax.lax.shift_right_logical(
        jax.lax.broadcasted_iota(jnp.int32, (_BB, _NC), 1), 4
    )
    for i_ref, r_ref, o_ref in ((i0, r0, o0), (i1, r1, o1), (i2, r2, o2), (i3, r3, o3)):
        mask = colv == i_ref[...]
        o_ref[...] = jnp.where(mask, r_ref[1:2, :], r_ref[0:1, :])


def kernel(input0, input1, input2, input3, table0, table1, table2, table3):
    inputs = (input0, input1, input2, input3)
    tables = (table0, table1, table2, table3)

    idx = [inp.reshape(_B, 1) for inp in inputs]
    rows = [jnp.stack([jnp.tile(t[0], _V), jnp.tile(t[1], _V)]) for t in tables]

    grid = (_B // _BB,)
    in_specs = (
        [pl.BlockSpec((_BB, 1), lambda i: (i, 0)) for _ in range(4)]
        + [pl.BlockSpec((2, _NC), lambda i: (0, 0)) for _ in range(4)]
    )
    out_specs = [pl.BlockSpec((_BB, _NC), lambda i: (i, 0)) for _ in range(4)]
    outs = pl.pallas_call(
        _fill_kernel,
        grid=grid,
        in_specs=in_specs,
        out_specs=out_specs,
        out_shape=[jax.ShapeDtypeStruct((_B, _NC), jnp.float32)] * 4,
    )(*idx, *rows)
    return tuple(o.reshape(_B, _V, _E) for o in outs)
